# Initial kernel scaffold; baseline (speedup 1.0000x reference)
import jax
import jax.numpy as jnp
from jax.experimental import pallas as pl


def kernel(cage, shape, shape_normals):
    raise NotImplementedError("write your pallas kernel here")


# speedup vs baseline: 1.0231x; 1.0231x over previous
import jax
import jax.numpy as jnp
from jax import lax
from jax.experimental import pallas as pl

INTERP = 10
EPSILON = 0.01

QT = 256
NT = 2048


def _loss_kernel(q_ref, s_ref, n_ref, out_ref):
    qb = q_ref[0]
    n_total = s_ref.shape[2]

    q0 = qb[:, 0:1]
    q1 = qb[:, 1:2]
    q2 = qb[:, 2:3]

    col = lax.broadcasted_iota(jnp.int32, (QT, NT), 1)

    def body(k, carry):
        run_min, run_dot = carry
        sl = pl.ds(k * NT, NT)
        s0 = s_ref[0, 0:1, sl]
        s1 = s_ref[0, 1:2, sl]
        s2 = s_ref[0, 2:3, sl]
        n0 = n_ref[0, 0:1, sl]
        n1 = n_ref[0, 1:2, sl]
        n2 = n_ref[0, 2:3, sl]

        d0 = q0 - s0
        d1 = q1 - s1
        d2c = q2 - s2
        dist = d0 * d0 + d1 * d1 + d2c * d2c
        dot = ((d0 - EPSILON * n0) * n0
               + (d1 - EPSILON * n1) * n1
               + (d2c - EPSILON * n2) * n2)

        mn = jnp.min(dist, axis=1, keepdims=True)
        idx = jnp.min(jnp.where(dist == mn, col, NT), axis=1, keepdims=True)
        dsel = jnp.sum(jnp.where(col == idx, dot, 0.0), axis=1, keepdims=True)

        upd = mn < run_min
        run_dot = jnp.where(upd, dsel, run_dot)
        run_min = jnp.where(upd, mn, run_min)
        return run_min, run_dot

    init = (jnp.full((QT, 1), jnp.inf, jnp.float32),
            jnp.zeros((QT, 1), jnp.float32))
    _, run_dot = lax.fori_loop(0, n_total // NT, body, init)

    loss = jnp.where(run_dot < 0.0, -run_dot, 0.0)
    part = jnp.sum(loss, axis=0, keepdims=True)

    i = pl.program_id(0)
    j = pl.program_id(1)
    first = jnp.logical_and(i == 0, j == 0)
    last = jnp.logical_and(i == pl.num_programs(0) - 1,
                           j == pl.num_programs(1) - 1)

    @pl.when(first)
    def _():
        out_ref[...] = jnp.zeros((1, 1), jnp.float32)

    out_ref[...] += part

    @pl.when(last)
    def _():
        out_ref[...] = out_ref[...] * (1.0 / (pl.num_programs(0)
                                              * pl.num_programs(1) * QT))


def kernel(cage, shape, shape_normals):
    b, m, d = cage.shape
    n = shape.shape[1]
    q_count = m * INTERP

    cage_p = jnp.concatenate([cage[:, 1:, :], cage[:, :1, :]], axis=1)
    t = jnp.linspace(0.0, 1.0, INTERP).reshape(1, 1, INTERP, 1)
    q = (t * cage_p[:, :, None, :]
         + (1.0 - t) * cage[:, :, None, :]).reshape(b, q_count, d)

    shape_t = shape.transpose(0, 2, 1)
    normals_t = shape_normals.transpose(0, 2, 1)

    out = pl.pallas_call(
        _loss_kernel,
        grid=(b, q_count // QT),
        in_specs=[
            pl.BlockSpec((1, QT, d), lambda i, j: (i, j, 0)),
            pl.BlockSpec((1, d, n), lambda i, j: (i, 0, 0)),
            pl.BlockSpec((1, d, n), lambda i, j: (i, 0, 0)),
        ],
        out_specs=pl.BlockSpec((1, 1), lambda i, j: (0, 0)),
        out_shape=jax.ShapeDtypeStruct((1, 1), jnp.float32),
    )(q, shape_t, normals_t)
    return out[0, 0]
